# Initial kernel scaffold; baseline (speedup 1.0000x reference)
import jax
import jax.numpy as jnp
from jax.experimental import pallas as pl


def kernel(feat, edge_index, W, attn):
    raise NotImplementedError("write your pallas kernel here")


# speedup vs baseline: 61.6922x; 61.6922x over previous
import functools

import jax
import jax.numpy as jnp
from jax import lax
from jax.experimental import pallas as pl
from jax.experimental.pallas import tpu as pltpu
from jax.experimental.pallas import tpu_sc as plsc

N = 10000
E = 320000
IN_FEATS = 128
H = 4
F = 32
HF = H * F
ROW = 144
NEG_SLOPE = 0.2

NC = 2
NS = 16
NW = NC * NS
EPW = E // NW
CHUNK = 80
NIT = EPW // CHUNK
RPT = N // NS

_f32 = jnp.float32


def _proj_body(feat_ref, wt_ref, c5_ref, c4_ref, hx_ref, er_ref):
    hb = jnp.dot(feat_ref[...], wt_ref[...], preferred_element_type=_f32, precision=lax.Precision.HIGHEST)
    extra = jnp.dot(hb, c5_ref[...], preferred_element_type=_f32, precision=lax.Precision.HIGHEST)
    hx_ref[...] = jnp.concatenate([hb, extra], axis=1)
    er_ref[...] = jnp.dot(hb, c4_ref[...], preferred_element_type=_f32, precision=lax.Precision.HIGHEST)


def _project(feat, Wt, C5, C4):
    bn = 400
    grid = (N // bn,)
    return pl.pallas_call(
        _proj_body,
        grid=grid,
        in_specs=[
            pl.BlockSpec((bn, IN_FEATS), lambda i: (i, 0)),
            pl.BlockSpec((IN_FEATS, HF), lambda i: (0, 0)),
            pl.BlockSpec((HF, 16), lambda i: (0, 0)),
            pl.BlockSpec((HF, 16), lambda i: (0, 0)),
        ],
        out_specs=[
            pl.BlockSpec((bn, ROW), lambda i: (i, 0)),
            pl.BlockSpec((bn, 16), lambda i: (i, 0)),
        ],
        out_shape=[
            jax.ShapeDtypeStruct((N, ROW), _f32),
            jax.ShapeDtypeStruct((N, 16), _f32),
        ],
    )(feat, Wt, C5, C4)


def _edge_body(hx_hbm, er_hbm, src_hbm, dst_hbm, out_hbm,
               srcb, dstb, erbuf, buf0, racc, gsem, esem):
    c = lax.axis_index("c")
    s = lax.axis_index("s")
    w = c * NS + s

    pltpu.sync_copy(src_hbm.at[w], srcb)
    pltpu.sync_copy(dst_hbm.at[w], dstb)

    zero16 = jnp.zeros((16,), _f32)

    def _zr(r, carry):
        for cc in range(ROW // 16):
            buf0[r, pl.ds(cc * 16, 16)] = zero16
        return carry

    lax.fori_loop(0, CHUNK, _zr, 0)
    base = s * RPT
    for k in range(RPT // CHUNK):
        pltpu.sync_copy(buf0, racc.at[pl.ds(base + k * CHUNK, CHUNK)])
    rem = RPT % CHUNK
    if rem:
        pltpu.sync_copy(buf0.at[pl.ds(0, rem)],
                        racc.at[pl.ds(base + RPT - rem, rem)])
    plsc.subcore_barrier()

    lane = lax.iota(jnp.int32, 16)

    def _iter(it, carry):
        gcp = pltpu.async_copy(hx_hbm.at[srcb.at[it]], buf0, gsem)
        ecp = pltpu.async_copy(er_hbm.at[dstb.at[it]], erbuf, esem)
        gcp.wait()
        ecp.wait()

        def _grp(g, carry2):
            ev = lane + g * 16
            for h in range(H):
                colw = jnp.full((16,), HF + h, jnp.int32)
                el = plsc.load_gather(buf0, [ev, colw])
                er = plsc.load_gather(erbuf, [ev, jnp.full((16,), h, jnp.int32)])
                e = el + er
                e = jnp.where(e > 0, e, _f32(NEG_SLOPE) * e)
                plsc.store_scatter(buf0, [ev, colw], jnp.exp(e))
            return carry2

        lax.fori_loop(0, CHUNK // 16, _grp, 0)

        def _edge(eix, carry2):
            wv = buf0[eix, pl.ds(HF, 16)]
            for h in range(H):
                wsc = wv[h]
                for j in range(2):
                    sl = pl.ds(h * F + j * 16, 16)
                    buf0[eix, sl] = buf0[eix, sl] * wsc
            return carry2

        lax.fori_loop(0, CHUNK, _edge, 0)

        pltpu.sync_copy(buf0, racc.at[dstb.at[it]], add=True)
        return carry

    lax.fori_loop(0, NIT, _iter, 0)

    plsc.subcore_barrier()
    pltpu.sync_copy(racc.at[pl.ds(base, RPT)], out_hbm.at[c, pl.ds(base, RPT)])


def _edge_pass(hx, ertab, src, dst):
    mesh = plsc.VectorSubcoreMesh(core_axis_name="c", subcore_axis_name="s")
    kern = pl.kernel(
        _edge_body,
        out_type=jax.ShapeDtypeStruct((NC, N, ROW), _f32),
        mesh=mesh,
        scratch_types=[
            pltpu.VMEM((NIT, CHUNK), jnp.int32),
            pltpu.VMEM((NIT, CHUNK), jnp.int32),
            pltpu.VMEM((CHUNK, 16), _f32),
            pltpu.VMEM((CHUNK, ROW), _f32),
            pltpu.VMEM_SHARED((N, ROW), _f32),
            pltpu.SemaphoreType.DMA,
            pltpu.SemaphoreType.DMA,
        ],
        compiler_params=pltpu.CompilerParams(
            use_tc_tiling_on_sc=False, needs_layout_passes=False),
    )
    return kern(hx, ertab, src, dst)


def _comb_body(p_ref, b2_ref, o_ref):
    rows = p_ref[0] + p_ref[1]
    dx = jnp.dot(rows, b2_ref[...], preferred_element_type=_f32, precision=lax.Precision.HIGHEST)
    o_ref[...] = rows[:, :HF] / (dx + _f32(1e-16))


def _combine(parts, B2):
    bn = 400
    grid = (N // bn,)
    return pl.pallas_call(
        _comb_body,
        grid=grid,
        in_specs=[
            pl.BlockSpec((NC, bn, ROW), lambda i: (0, i, 0)),
            pl.BlockSpec((ROW, HF), lambda i: (0, 0)),
        ],
        out_specs=pl.BlockSpec((bn, HF), lambda i: (i, 0)),
        out_shape=jax.ShapeDtypeStruct((N, HF), _f32),
    )(parts, B2)


@jax.jit
def kernel(feat, edge_index, W, attn):
    Wt = W.T

    a = attn[0]
    a_l = a[:, :F]
    a_r = a[:, F:]
    C5 = jnp.zeros((HF, 16), _f32)
    C4 = jnp.zeros((HF, 16), _f32)
    for h in range(H):
        C5 = C5.at[h * F:(h + 1) * F, h].set(a_l[h])
        C4 = C4.at[h * F:(h + 1) * F, h].set(a_r[h])
    B2 = jnp.zeros((ROW, HF), _f32)
    for h in range(H):
        B2 = B2.at[HF + h, h * F:(h + 1) * F].set(1.0)

    hx, ertab = _project(feat, Wt, C5, C4)

    src = edge_index[0].reshape(NW, NIT, CHUNK)
    dst = edge_index[1].reshape(NW, NIT, CHUNK)
    parts = _edge_pass(hx, ertab, src, dst)

    out = _combine(parts, B2)
    return out.reshape(N, H, F)


# speedup vs baseline: 86.9870x; 1.4100x over previous
import functools

import jax
import jax.numpy as jnp
from jax import lax
from jax.experimental import pallas as pl
from jax.experimental.pallas import tpu as pltpu
from jax.experimental.pallas import tpu_sc as plsc

N = 10000
E = 320000
IN_FEATS = 128
H = 4
F = 32
HF = H * F
ROW = 144
NEG_SLOPE = 0.2

NC = 2
NS = 16
NW = NC * NS
EPW = E // NW
CHUNK = 80
NIT = EPW // CHUNK
RPT = N // NS

_f32 = jnp.float32


def _proj_body(feat_ref, wt_ref, c5_ref, c4_ref, hx_ref, er_ref):
    hb = jnp.dot(feat_ref[...], wt_ref[...], preferred_element_type=_f32, precision=lax.Precision.HIGHEST)
    extra = jnp.dot(hb, c5_ref[...], preferred_element_type=_f32, precision=lax.Precision.HIGHEST)
    hx_ref[...] = jnp.concatenate([hb, extra], axis=1)
    er_ref[...] = jnp.dot(hb, c4_ref[...], preferred_element_type=_f32, precision=lax.Precision.HIGHEST)


def _project(feat, Wt, C5, C4):
    bn = 400
    grid = (N // bn,)
    return pl.pallas_call(
        _proj_body,
        grid=grid,
        in_specs=[
            pl.BlockSpec((bn, IN_FEATS), lambda i: (i, 0)),
            pl.BlockSpec((IN_FEATS, HF), lambda i: (0, 0)),
            pl.BlockSpec((HF, 16), lambda i: (0, 0)),
            pl.BlockSpec((HF, 16), lambda i: (0, 0)),
        ],
        out_specs=[
            pl.BlockSpec((bn, ROW), lambda i: (i, 0)),
            pl.BlockSpec((bn, 16), lambda i: (i, 0)),
        ],
        out_shape=[
            jax.ShapeDtypeStruct((N, ROW), _f32),
            jax.ShapeDtypeStruct((N, 16), _f32),
        ],
    )(feat, Wt, C5, C4)


SUP = 25
NSUP = NIT // SUP


def _edge_body(hx_hbm, er_hbm, src_hbm, dst_hbm, out_hbm,
               srcb, dstb, erb0, erb1, buf0, buf1, racc,
               gsem0, gsem1, esem0, esem1):
    c = lax.axis_index("c")
    s = lax.axis_index("s")
    w = c * NS + s

    zero16 = jnp.zeros((16,), _f32)

    def _zr(r, carry):
        for cc in range(ROW // 16):
            buf0[r, pl.ds(cc * 16, 16)] = zero16
        return carry

    lax.fori_loop(0, CHUNK, _zr, 0)
    base = s * RPT
    for k in range(RPT // CHUNK):
        pltpu.sync_copy(buf0, racc.at[pl.ds(base + k * CHUNK, CHUNK)])
    rem = RPT % CHUNK
    if rem:
        pltpu.sync_copy(buf0.at[pl.ds(0, rem)],
                        racc.at[pl.ds(base + RPT - rem, rem)])
    plsc.subcore_barrier()

    lane = lax.iota(jnp.int32, 16)
    bufs = (buf0, buf1)
    erbs = (erb0, erb1)
    gsems = (gsem0, gsem1)
    esems = (esem0, esem1)

    def _issue(itl, b):
        pltpu.async_copy(hx_hbm.at[srcb.at[itl]], bufs[b], gsems[b])
        pltpu.async_copy(er_hbm.at[dstb.at[itl]], erbs[b], esems[b])

    def _wait(b):
        pltpu.make_async_copy(hx_hbm.at[srcb.at[0]], bufs[b], gsems[b]).wait()
        pltpu.make_async_copy(er_hbm.at[dstb.at[0]], erbs[b], esems[b]).wait()

    def _compute(itl, b):
        buf = bufs[b]
        erb = erbs[b]

        def _grp(g, carry2):
            ev = lane + g * 16
            for h in range(H):
                colw = jnp.full((16,), HF + h, jnp.int32)
                el = plsc.load_gather(buf, [ev, colw])
                er = plsc.load_gather(erb, [ev, jnp.full((16,), h, jnp.int32)])
                e = el + er
                e = jnp.where(e > 0, e, _f32(NEG_SLOPE) * e)
                plsc.store_scatter(buf, [ev, colw], jnp.exp(e))
            return carry2

        lax.fori_loop(0, CHUNK // 16, _grp, 0)

        def _edge(eix, carry2):
            wv = buf[eix, pl.ds(HF, 16)]
            for h in range(H):
                wsc = wv[h]
                for j in range(2):
                    sl = pl.ds(h * F + j * 16, 16)
                    buf[eix, sl] = buf[eix, sl] * wsc
            return carry2

        lax.fori_loop(0, CHUNK, _edge, 0)

        pltpu.sync_copy(buf, racc.at[dstb.at[itl]], add=True)

    def _sup(sp, carry):
        pltpu.sync_copy(src_hbm.at[w, pl.ds(sp * SUP, SUP)], srcb)
        pltpu.sync_copy(dst_hbm.at[w, pl.ds(sp * SUP, SUP)], dstb)
        _issue(0, 0)
        _issue(1, 1)

        def _pair(j2, carry2):
            for b in range(2):
                itl = j2 * 2 + b
                _wait(b)
                _compute(itl, b)

                @pl.when(itl + 2 < SUP)
                def _():
                    _issue(itl + 2, b)
            return carry2

        lax.fori_loop(0, SUP // 2, _pair, 0)
        _wait((SUP - 1) % 2)
        _compute(SUP - 1, (SUP - 1) % 2)
        return carry

    lax.fori_loop(0, NSUP, _sup, 0)

    plsc.subcore_barrier()
    pltpu.sync_copy(racc.at[pl.ds(base, RPT)], out_hbm.at[c, pl.ds(base, RPT)])


def _edge_pass(hx, ertab, src, dst):
    mesh = plsc.VectorSubcoreMesh(core_axis_name="c", subcore_axis_name="s")
    kern = pl.kernel(
        _edge_body,
        out_type=jax.ShapeDtypeStruct((NC, N, ROW), _f32),
        mesh=mesh,
        scratch_types=[
            pltpu.VMEM((SUP, CHUNK), jnp.int32),
            pltpu.VMEM((SUP, CHUNK), jnp.int32),
            pltpu.VMEM((CHUNK, 16), _f32),
            pltpu.VMEM((CHUNK, 16), _f32),
            pltpu.VMEM((CHUNK, ROW), _f32),
            pltpu.VMEM((CHUNK, ROW), _f32),
            pltpu.VMEM_SHARED((N, ROW), _f32),
            pltpu.SemaphoreType.DMA,
            pltpu.SemaphoreType.DMA,
            pltpu.SemaphoreType.DMA,
            pltpu.SemaphoreType.DMA,
        ],
        compiler_params=pltpu.CompilerParams(
            use_tc_tiling_on_sc=False, needs_layout_passes=False),
    )
    return kern(hx, ertab, src, dst)


def _comb_body(p_ref, b2_ref, o_ref):
    rows = p_ref[0] + p_ref[1]
    dx = jnp.dot(rows, b2_ref[...], preferred_element_type=_f32, precision=lax.Precision.HIGHEST)
    o_ref[...] = rows[:, :HF] / (dx + _f32(1e-16))


def _combine(parts, B2):
    bn = 400
    grid = (N // bn,)
    return pl.pallas_call(
        _comb_body,
        grid=grid,
        in_specs=[
            pl.BlockSpec((NC, bn, ROW), lambda i: (0, i, 0)),
            pl.BlockSpec((ROW, HF), lambda i: (0, 0)),
        ],
        out_specs=pl.BlockSpec((bn, HF), lambda i: (i, 0)),
        out_shape=jax.ShapeDtypeStruct((N, HF), _f32),
    )(parts, B2)


@jax.jit
def kernel(feat, edge_index, W, attn):
    Wt = W.T

    a = attn[0]
    a_l = a[:, :F]
    a_r = a[:, F:]
    C5 = jnp.zeros((HF, 16), _f32)
    C4 = jnp.zeros((HF, 16), _f32)
    for h in range(H):
        C5 = C5.at[h * F:(h + 1) * F, h].set(a_l[h])
        C4 = C4.at[h * F:(h + 1) * F, h].set(a_r[h])
    B2 = jnp.zeros((ROW, HF), _f32)
    for h in range(H):
        B2 = B2.at[HF + h, h * F:(h + 1) * F].set(1.0)

    hx, ertab = _project(feat, Wt, C5, C4)

    src = edge_index[0].reshape(NW, NIT, CHUNK)
    dst = edge_index[1].reshape(NW, NIT, CHUNK)
    parts = _edge_pass(hx, ertab, src, dst)

    out = _combine(parts, B2)
    return out.reshape(N, H, F)


# speedup vs baseline: 88.1468x; 1.0133x over previous
import functools

import jax
import jax.numpy as jnp
from jax import lax
from jax.experimental import pallas as pl
from jax.experimental.pallas import tpu as pltpu
from jax.experimental.pallas import tpu_sc as plsc

N = 10000
E = 320000
IN_FEATS = 128
H = 4
F = 32
HF = H * F
ROW = 144
NEG_SLOPE = 0.2

NC = 2
NS = 16
NW = NC * NS
EPW = E // NW
CHUNK = 80
NIT = EPW // CHUNK
RPT = N // NS

_f32 = jnp.float32


def _proj_body(feat_ref, wt_ref, c5_ref, c4_ref, hx_ref, er_ref):
    hb = jnp.dot(feat_ref[...], wt_ref[...], preferred_element_type=_f32, precision=lax.Precision.HIGHEST)
    extra = jnp.dot(hb, c5_ref[...], preferred_element_type=_f32, precision=lax.Precision.HIGHEST)
    hx_ref[...] = jnp.concatenate([hb, extra], axis=1)
    er_ref[...] = jnp.dot(hb, c4_ref[...], preferred_element_type=_f32, precision=lax.Precision.HIGHEST)


def _project(feat, Wt, C5, C4):
    bn = 400
    grid = (N // bn,)
    return pl.pallas_call(
        _proj_body,
        grid=grid,
        in_specs=[
            pl.BlockSpec((bn, IN_FEATS), lambda i: (i, 0)),
            pl.BlockSpec((IN_FEATS, HF), lambda i: (0, 0)),
            pl.BlockSpec((HF, 16), lambda i: (0, 0)),
            pl.BlockSpec((HF, 16), lambda i: (0, 0)),
        ],
        out_specs=[
            pl.BlockSpec((bn, ROW), lambda i: (i, 0)),
            pl.BlockSpec((bn, 16), lambda i: (i, 0)),
        ],
        out_shape=[
            jax.ShapeDtypeStruct((N, ROW), _f32),
            jax.ShapeDtypeStruct((N, 16), _f32),
        ],
    )(feat, Wt, C5, C4)


SUP = 25
NSUP = NIT // SUP


def _edge_body(hx_hbm, er_hbm, src_hbm, dst_hbm, out_hbm,
               srcb, dstb, erb0, erb1, buf0, buf1, racc,
               gsem0, gsem1, esem0, esem1):
    c = lax.axis_index("c")
    s = lax.axis_index("s")
    w = c * NS + s

    zero16 = jnp.zeros((16,), _f32)

    def _zr(r, carry):
        for cc in range(ROW // 16):
            buf0[r, pl.ds(cc * 16, 16)] = zero16
        return carry

    lax.fori_loop(0, CHUNK, _zr, 0)
    base = s * RPT
    for k in range(RPT // CHUNK):
        pltpu.sync_copy(buf0, racc.at[pl.ds(base + k * CHUNK, CHUNK)])
    rem = RPT % CHUNK
    if rem:
        pltpu.sync_copy(buf0.at[pl.ds(0, rem)],
                        racc.at[pl.ds(base + RPT - rem, rem)])
    plsc.subcore_barrier()

    lane = lax.iota(jnp.int32, 16)
    bufs = (buf0, buf1)
    erbs = (erb0, erb1)
    gsems = (gsem0, gsem1)
    esems = (esem0, esem1)

    def _issue(itl, b):
        pltpu.async_copy(hx_hbm.at[srcb.at[itl]], bufs[b], gsems[b])
        pltpu.async_copy(er_hbm.at[dstb.at[itl]], erbs[b], esems[b])

    def _wait(b):
        pltpu.make_async_copy(hx_hbm.at[srcb.at[0]], bufs[b], gsems[b]).wait()
        pltpu.make_async_copy(er_hbm.at[dstb.at[0]], erbs[b], esems[b]).wait()

    def _compute(itl, b):
        buf = bufs[b]
        erb = erbs[b]

        @plsc.parallel_loop(0, CHUNK // 16, unroll=5)
        def _grp(g):
            ev = lane + g * 16
            for h in range(H):
                colw = jnp.full((16,), HF + h, jnp.int32)
                el = plsc.load_gather(buf, [ev, colw])
                er = plsc.load_gather(erb, [ev, jnp.full((16,), h, jnp.int32)])
                e = el + er
                e = jnp.where(e > 0, e, _f32(NEG_SLOPE) * e)
                plsc.store_scatter(buf, [ev, colw], jnp.exp(e))

        @plsc.parallel_loop(0, CHUNK, unroll=4)
        def _edge(eix):
            wv = buf[eix, pl.ds(HF, 16)]
            for h in range(H):
                wsc = wv[h]
                for j in range(2):
                    sl = pl.ds(h * F + j * 16, 16)
                    buf[eix, sl] = buf[eix, sl] * wsc

        pltpu.sync_copy(buf, racc.at[dstb.at[itl]], add=True)

    def _sup(sp, carry):
        pltpu.sync_copy(src_hbm.at[w, pl.ds(sp * SUP, SUP)], srcb)
        pltpu.sync_copy(dst_hbm.at[w, pl.ds(sp * SUP, SUP)], dstb)
        _issue(0, 0)
        _issue(1, 1)

        def _pair(j2, carry2):
            for b in range(2):
                itl = j2 * 2 + b
                _wait(b)
                _compute(itl, b)

                @pl.when(itl + 2 < SUP)
                def _():
                    _issue(itl + 2, b)
            return carry2

        lax.fori_loop(0, SUP // 2, _pair, 0)
        _wait((SUP - 1) % 2)
        _compute(SUP - 1, (SUP - 1) % 2)
        return carry

    lax.fori_loop(0, NSUP, _sup, 0)

    plsc.subcore_barrier()
    pltpu.sync_copy(racc.at[pl.ds(base, RPT)], out_hbm.at[c, pl.ds(base, RPT)])


def _edge_pass(hx, ertab, src, dst):
    mesh = plsc.VectorSubcoreMesh(core_axis_name="c", subcore_axis_name="s")
    kern = pl.kernel(
        _edge_body,
        out_type=jax.ShapeDtypeStruct((NC, N, ROW), _f32),
        mesh=mesh,
        scratch_types=[
            pltpu.VMEM((SUP, CHUNK), jnp.int32),
            pltpu.VMEM((SUP, CHUNK), jnp.int32),
            pltpu.VMEM((CHUNK, 16), _f32),
            pltpu.VMEM((CHUNK, 16), _f32),
            pltpu.VMEM((CHUNK, ROW), _f32),
            pltpu.VMEM((CHUNK, ROW), _f32),
            pltpu.VMEM_SHARED((N, ROW), _f32),
            pltpu.SemaphoreType.DMA,
            pltpu.SemaphoreType.DMA,
            pltpu.SemaphoreType.DMA,
            pltpu.SemaphoreType.DMA,
        ],
        compiler_params=pltpu.CompilerParams(
            use_tc_tiling_on_sc=False, needs_layout_passes=False),
    )
    return kern(hx, ertab, src, dst)


def _comb_body(p_ref, b2_ref, o_ref):
    rows = p_ref[0] + p_ref[1]
    dx = jnp.dot(rows, b2_ref[...], preferred_element_type=_f32, precision=lax.Precision.HIGHEST)
    o_ref[...] = rows[:, :HF] / (dx + _f32(1e-16))


def _combine(parts, B2):
    bn = 400
    grid = (N // bn,)
    return pl.pallas_call(
        _comb_body,
        grid=grid,
        in_specs=[
            pl.BlockSpec((NC, bn, ROW), lambda i: (0, i, 0)),
            pl.BlockSpec((ROW, HF), lambda i: (0, 0)),
        ],
        out_specs=pl.BlockSpec((bn, HF), lambda i: (i, 0)),
        out_shape=jax.ShapeDtypeStruct((N, HF), _f32),
    )(parts, B2)


@jax.jit
def kernel(feat, edge_index, W, attn):
    Wt = W.T

    a = attn[0]
    a_l = a[:, :F]
    a_r = a[:, F:]
    C5 = jnp.zeros((HF, 16), _f32)
    C4 = jnp.zeros((HF, 16), _f32)
    for h in range(H):
        C5 = C5.at[h * F:(h + 1) * F, h].set(a_l[h])
        C4 = C4.at[h * F:(h + 1) * F, h].set(a_r[h])
    B2 = jnp.zeros((ROW, HF), _f32)
    for h in range(H):
        B2 = B2.at[HF + h, h * F:(h + 1) * F].set(1.0)

    hx, ertab = _project(feat, Wt, C5, C4)

    src = edge_index[0].reshape(NW, NIT, CHUNK)
    dst = edge_index[1].reshape(NW, NIT, CHUNK)
    parts = _edge_pass(hx, ertab, src, dst)

    out = _combine(parts, B2)
    return out.reshape(N, H, F)
